# baseline (speedup 1.0000x reference)
import functools

import jax
import jax.numpy as jnp
from jax.experimental import pallas as pl
from jax.experimental.pallas import tpu as pltpu

P, L = 10000, 2048
DP, DL, DO = 128, 128, 128
TP = 2000
EPS = 1e-5


def _mega_kernel(a0_ref, a1_ref, px_ref, lx_ref, wpl1_ref, wpl2_ref, bpl_ref,
                 wlp1_ref, wlp2_ref, blp_ref, glp_ref, belp_ref,
                 gpl_ref, bepl_ref,
                 pxp_ref, lxp_ref,
                 h_scr, mlpT_scr, stats_scr):
    i = pl.program_id(0)
    ns = pl.num_programs(0)
    LH = L // 2

    a0_bf = a0_ref[...].astype(jnp.bfloat16)
    a1_bf = a1_ref[...].astype(jnp.bfloat16)
    pxt = px_ref[...]
    pxt_bf = pxt.astype(jnp.bfloat16)
    lx0_bf = lx_ref[:LH, :].astype(jnp.bfloat16)
    lx1_bf = lx_ref[LH:, :].astype(jnp.bfloat16)

    mpl = (jnp.dot(a0_bf, lx0_bf, preferred_element_type=jnp.float32)
           + jnp.dot(a1_bf, lx1_bf, preferred_element_type=jnp.float32))
    h = (jnp.dot(mpl, wpl1_ref[...], preferred_element_type=jnp.float32)
         + jnp.dot(pxt, wpl2_ref[...], preferred_element_type=jnp.float32)
         + bpl_ref[...])
    h_scr[pl.ds(i * TP, TP), :] = h.astype(jnp.bfloat16)

    ssum = jnp.sum(h, axis=0, keepdims=True)
    ssq = jnp.sum(h * h, axis=0, keepdims=True)
    st = jnp.concatenate([ssum, ssq], axis=0)

    mlpT0 = jax.lax.dot_general(
        pxt_bf, a0_bf, (((0,), (0,)), ((), ())),
        preferred_element_type=jnp.float32)
    mlpT1 = jax.lax.dot_general(
        pxt_bf, a1_bf, (((0,), (0,)), ((), ())),
        preferred_element_type=jnp.float32)

    @pl.when(i == 0)
    def _init():
        mlpT_scr[:, :LH] = mlpT0
        mlpT_scr[:, LH:] = mlpT1
        stats_scr[...] = st

    @pl.when(i > 0)
    def _acc():
        mlpT_scr[:, :LH] += mlpT0
        mlpT_scr[:, LH:] += mlpT1
        stats_scr[...] += st

    @pl.when(i == ns - 1)
    def _final():
        hl = (jax.lax.dot_general(mlpT_scr[...], wlp1_ref[...],
                                  (((0,), (0,)), ((), ())),
                                  preferred_element_type=jnp.float32)
              + jnp.dot(lx_ref[...], wlp2_ref[...],
                        preferred_element_type=jnp.float32)
              + blp_ref[...])
        meanl = jnp.mean(hl, axis=0, keepdims=True)
        varl = jnp.mean((hl - meanl) ** 2, axis=0, keepdims=True)
        hnl = (hl - meanl) * jax.lax.rsqrt(varl + EPS)
        pxp_ref[...] = jnp.maximum(hnl * glp_ref[...] + belp_ref[...], 0.0)

        n = jnp.float32(P)
        mean = stats_scr[0:1, :] / n
        var = stats_scr[1:2, :] / n - mean * mean
        scale = gpl_ref[...] * jax.lax.rsqrt(var + EPS)
        shift = bepl_ref[...] - mean * scale
        hp = h_scr[...].astype(jnp.float32)
        lxp_ref[...] = jnp.maximum(hp * scale + shift, 0.0)


@functools.partial(jax.jit, static_argnames=())
def kernel(px, lx, pl_mat, W_lp, b_lp, g_lp, be_lp, W_pl, b_pl, g_pl, be_pl):
    wpl1 = W_pl[:, :DL].T
    wpl2 = W_pl[:, DL:].T
    wlp1 = W_lp[:, :DP].T
    wlp2 = W_lp[:, DP:].T
    b_pl2 = b_pl.reshape(1, DO)
    b_lp2 = b_lp.reshape(1, DO)
    g_pl2 = g_pl.reshape(1, DO)
    be_pl2 = be_pl.reshape(1, DO)
    g_lp2 = g_lp.reshape(1, DO)
    be_lp2 = be_lp.reshape(1, DO)

    ns = P // TP
    px_p, lx_p = pl.pallas_call(
        _mega_kernel,
        grid=(ns,),
        in_specs=[
            pl.BlockSpec((TP, L // 2), lambda i: (i, 0)),
            pl.BlockSpec((TP, L // 2), lambda i: (i, 1)),
            pl.BlockSpec((TP, DP), lambda i: (i, 0)),
            pl.BlockSpec((L, DL), lambda i: (0, 0)),
            pl.BlockSpec((DL, DO), lambda i: (0, 0)),
            pl.BlockSpec((DP, DO), lambda i: (0, 0)),
            pl.BlockSpec((1, DO), lambda i: (0, 0)),
            pl.BlockSpec((DP, DO), lambda i: (0, 0)),
            pl.BlockSpec((DL, DO), lambda i: (0, 0)),
            pl.BlockSpec((1, DO), lambda i: (0, 0)),
            pl.BlockSpec((1, DO), lambda i: (0, 0)),
            pl.BlockSpec((1, DO), lambda i: (0, 0)),
            pl.BlockSpec((1, DO), lambda i: (0, 0)),
            pl.BlockSpec((1, DO), lambda i: (0, 0)),
        ],
        out_specs=[
            pl.BlockSpec((L, DO), lambda i: (0, 0)),
            pl.BlockSpec((P, DO), lambda i: (0, 0)),
        ],
        out_shape=[
            jax.ShapeDtypeStruct((L, DO), jnp.float32),
            jax.ShapeDtypeStruct((P, DO), jnp.float32),
        ],
        scratch_shapes=[
            pltpu.VMEM((P, DO), jnp.bfloat16),
            pltpu.VMEM((DP, L), jnp.float32),
            pltpu.VMEM((2, DO), jnp.float32),
        ],
    )(pl_mat, pl_mat, px, lx, wpl1, wpl2, b_pl2,
      wlp1, wlp2, b_lp2, g_lp2, be_lp2, g_pl2, be_pl2)

    return (px_p, lx_p)


# speedup vs baseline: 1.1542x; 1.1542x over previous
import functools

import jax
import jax.numpy as jnp
from jax.experimental import pallas as pl
from jax.experimental.pallas import tpu as pltpu

P, L = 10000, 2048
DP, DL, DO = 128, 128, 128
TP = 2000
EPS = 1e-5


def _mega_kernel(a_ref, px_ref, lx_ref, wpl1_ref, wpl2_ref, bpl_ref,
                 wlp1_ref, wlp2_ref, blp_ref, glp_ref, belp_ref,
                 gpl_ref, bepl_ref,
                 pxp_ref, lxp_ref,
                 h_scr, mlpT_scr, stats_scr, lxw_scr):
    i = pl.program_id(0)
    ns = pl.num_programs(0)

    a_bf = a_ref[...].astype(jnp.bfloat16)
    pxt = px_ref[...]
    pxt_bf = pxt.astype(jnp.bfloat16)

    @pl.when(i == 0)
    def _fold():
        lx_bf = lx_ref[...].astype(jnp.bfloat16)
        lxw_scr[...] = jnp.dot(
            lx_bf, wpl1_ref[...].astype(jnp.bfloat16),
            preferred_element_type=jnp.float32).astype(jnp.bfloat16)

    h = (jnp.dot(a_bf, lxw_scr[...], preferred_element_type=jnp.float32)
         + jnp.dot(pxt_bf, wpl2_ref[...].astype(jnp.bfloat16),
                   preferred_element_type=jnp.float32)
         + bpl_ref[...])
    h_scr[pl.ds(i * TP, TP), :] = h.astype(jnp.bfloat16)

    ssum = jnp.sum(h, axis=0, keepdims=True)
    ssq = jnp.sum(h * h, axis=0, keepdims=True)
    st = jnp.concatenate([ssum, ssq], axis=0)

    mlpT_part = jax.lax.dot_general(
        pxt_bf, a_bf, (((0,), (0,)), ((), ())),
        preferred_element_type=jnp.float32)

    @pl.when(i == 0)
    def _init():
        mlpT_scr[...] = mlpT_part
        stats_scr[...] = st

    @pl.when(i > 0)
    def _acc():
        mlpT_scr[...] += mlpT_part
        stats_scr[...] += st

    @pl.when(i == ns - 1)
    def _final():
        hl = (jax.lax.dot_general(mlpT_scr[...], wlp1_ref[...],
                                  (((0,), (0,)), ((), ())),
                                  preferred_element_type=jnp.float32)
              + jnp.dot(lx_ref[...], wlp2_ref[...],
                        preferred_element_type=jnp.float32)
              + blp_ref[...])
        meanl = jnp.mean(hl, axis=0, keepdims=True)
        varl = jnp.mean((hl - meanl) ** 2, axis=0, keepdims=True)
        hnl = (hl - meanl) * jax.lax.rsqrt(varl + EPS)
        pxp_ref[...] = jnp.maximum(hnl * glp_ref[...] + belp_ref[...], 0.0)

        n = jnp.float32(P)
        mean = stats_scr[0:1, :] / n
        var = stats_scr[1:2, :] / n - mean * mean
        scale = gpl_ref[...] * jax.lax.rsqrt(var + EPS)
        shift = bepl_ref[...] - mean * scale
        hp = h_scr[...].astype(jnp.float32)
        lxp_ref[...] = jnp.maximum(hp * scale + shift, 0.0)


@functools.partial(jax.jit, static_argnames=())
def kernel(px, lx, pl_mat, W_lp, b_lp, g_lp, be_lp, W_pl, b_pl, g_pl, be_pl):
    wpl1 = W_pl[:, :DL].T
    wpl2 = W_pl[:, DL:].T
    wlp1 = W_lp[:, :DP].T
    wlp2 = W_lp[:, DP:].T
    b_pl2 = b_pl.reshape(1, DO)
    b_lp2 = b_lp.reshape(1, DO)
    g_pl2 = g_pl.reshape(1, DO)
    be_pl2 = be_pl.reshape(1, DO)
    g_lp2 = g_lp.reshape(1, DO)
    be_lp2 = be_lp.reshape(1, DO)

    ns = P // TP
    px_p, lx_p = pl.pallas_call(
        _mega_kernel,
        grid=(ns,),
        in_specs=[
            pl.BlockSpec((TP, L), lambda i: (i, 0)),
            pl.BlockSpec((TP, DP), lambda i: (i, 0)),
            pl.BlockSpec((L, DL), lambda i: (0, 0)),
            pl.BlockSpec((DL, DO), lambda i: (0, 0)),
            pl.BlockSpec((DP, DO), lambda i: (0, 0)),
            pl.BlockSpec((1, DO), lambda i: (0, 0)),
            pl.BlockSpec((DP, DO), lambda i: (0, 0)),
            pl.BlockSpec((DL, DO), lambda i: (0, 0)),
            pl.BlockSpec((1, DO), lambda i: (0, 0)),
            pl.BlockSpec((1, DO), lambda i: (0, 0)),
            pl.BlockSpec((1, DO), lambda i: (0, 0)),
            pl.BlockSpec((1, DO), lambda i: (0, 0)),
            pl.BlockSpec((1, DO), lambda i: (0, 0)),
        ],
        out_specs=[
            pl.BlockSpec((L, DO), lambda i: (0, 0)),
            pl.BlockSpec((P, DO), lambda i: (0, 0)),
        ],
        out_shape=[
            jax.ShapeDtypeStruct((L, DO), jnp.float32),
            jax.ShapeDtypeStruct((P, DO), jnp.float32),
        ],
        scratch_shapes=[
            pltpu.VMEM((P, DO), jnp.bfloat16),
            pltpu.VMEM((DP, L), jnp.float32),
            pltpu.VMEM((2, DO), jnp.float32),
            pltpu.VMEM((L, DO), jnp.bfloat16),
        ],
    )(pl_mat, px, lx, wpl1, wpl2, b_pl2,
      wlp1, wlp2, b_lp2, g_lp2, be_lp2, g_pl2, be_pl2)

    return (px_p, lx_p)


# speedup vs baseline: 1.1700x; 1.0137x over previous
import functools

import jax
import jax.numpy as jnp
from jax.experimental import pallas as pl
from jax.experimental.pallas import tpu as pltpu

P, L = 10000, 2048
DP, DL, DO = 128, 128, 128
TP = 2000
NS = P // TP
EPS = 1e-5


def _mega_kernel(a_ref, px_ref, lx_ref, wpl1_ref, wpl2_ref, bpl_ref,
                 wlp1_ref, wlp2_ref, blp_ref, glp_ref, belp_ref,
                 gpl_ref, bepl_ref,
                 pxp_ref, lxp_ref,
                 h_scr, mlpT_scr, stats_scr, lxw_scr):
    i = pl.program_id(0)
    ns = pl.num_programs(0)

    a_bf = a_ref[...].astype(jnp.bfloat16)
    pxt = px_ref[...]
    pxt_bf = pxt.astype(jnp.bfloat16)

    @pl.when(i == 0)
    def _fold():
        lx_bf = lx_ref[...].astype(jnp.bfloat16)
        lxw_scr[...] = jnp.dot(
            lx_bf, wpl1_ref[...].astype(jnp.bfloat16),
            preferred_element_type=jnp.float32).astype(jnp.bfloat16)

    h = (jnp.dot(a_bf, lxw_scr[...], preferred_element_type=jnp.float32)
         + jnp.dot(pxt_bf, wpl2_ref[...].astype(jnp.bfloat16),
                   preferred_element_type=jnp.float32)
         + bpl_ref[...])
    h_scr[pl.ds(i * TP, TP), :] = h.astype(jnp.bfloat16)

    ssum = jnp.sum(h, axis=0, keepdims=True)
    ssq = jnp.sum(h * h, axis=0, keepdims=True)
    st = jnp.concatenate([ssum, ssq], axis=0)

    mlpT_part = jax.lax.dot_general(
        pxt_bf, a_bf, (((0,), (0,)), ((), ())),
        preferred_element_type=jnp.float32)
    mlpT_scr[i] = mlpT_part

    @pl.when(i == 0)
    def _init():
        stats_scr[...] = st

    @pl.when(i > 0)
    def _acc():
        stats_scr[...] += st

    @pl.when(i == ns - 1)
    def _final():
        mlpT = mlpT_scr[0]
        for k in range(1, NS):
            mlpT = mlpT + mlpT_scr[k]
        hl = (jax.lax.dot_general(mlpT, wlp1_ref[...],
                                  (((0,), (0,)), ((), ())),
                                  preferred_element_type=jnp.float32)
              + jnp.dot(lx_ref[...], wlp2_ref[...],
                        preferred_element_type=jnp.float32)
              + blp_ref[...])
        meanl = jnp.mean(hl, axis=0, keepdims=True)
        varl = jnp.mean((hl - meanl) ** 2, axis=0, keepdims=True)
        hnl = (hl - meanl) * jax.lax.rsqrt(varl + EPS)
        pxp_ref[...] = jnp.maximum(hnl * glp_ref[...] + belp_ref[...], 0.0)

        n = jnp.float32(P)
        mean = stats_scr[0:1, :] / n
        var = stats_scr[1:2, :] / n - mean * mean
        scale = gpl_ref[...] * jax.lax.rsqrt(var + EPS)
        shift = bepl_ref[...] - mean * scale
        hp = h_scr[...].astype(jnp.float32)
        lxp_ref[...] = jnp.maximum(hp * scale + shift, 0.0)


@functools.partial(jax.jit, static_argnames=())
def kernel(px, lx, pl_mat, W_lp, b_lp, g_lp, be_lp, W_pl, b_pl, g_pl, be_pl):
    wpl1 = W_pl[:, :DL].T
    wpl2 = W_pl[:, DL:].T
    wlp1 = W_lp[:, :DP].T
    wlp2 = W_lp[:, DP:].T
    b_pl2 = b_pl.reshape(1, DO)
    b_lp2 = b_lp.reshape(1, DO)
    g_pl2 = g_pl.reshape(1, DO)
    be_pl2 = be_pl.reshape(1, DO)
    g_lp2 = g_lp.reshape(1, DO)
    be_lp2 = be_lp.reshape(1, DO)

    ns = P // TP
    px_p, lx_p = pl.pallas_call(
        _mega_kernel,
        grid=(ns,),
        in_specs=[
            pl.BlockSpec((TP, L), lambda i: (i, 0)),
            pl.BlockSpec((TP, DP), lambda i: (i, 0)),
            pl.BlockSpec((L, DL), lambda i: (0, 0)),
            pl.BlockSpec((DL, DO), lambda i: (0, 0)),
            pl.BlockSpec((DP, DO), lambda i: (0, 0)),
            pl.BlockSpec((1, DO), lambda i: (0, 0)),
            pl.BlockSpec((DP, DO), lambda i: (0, 0)),
            pl.BlockSpec((DL, DO), lambda i: (0, 0)),
            pl.BlockSpec((1, DO), lambda i: (0, 0)),
            pl.BlockSpec((1, DO), lambda i: (0, 0)),
            pl.BlockSpec((1, DO), lambda i: (0, 0)),
            pl.BlockSpec((1, DO), lambda i: (0, 0)),
            pl.BlockSpec((1, DO), lambda i: (0, 0)),
        ],
        out_specs=[
            pl.BlockSpec((L, DO), lambda i: (0, 0)),
            pl.BlockSpec((P, DO), lambda i: (0, 0)),
        ],
        out_shape=[
            jax.ShapeDtypeStruct((L, DO), jnp.float32),
            jax.ShapeDtypeStruct((P, DO), jnp.float32),
        ],
        scratch_shapes=[
            pltpu.VMEM((P, DO), jnp.bfloat16),
            pltpu.VMEM((NS, DP, L), jnp.float32),
            pltpu.VMEM((2, DO), jnp.float32),
            pltpu.VMEM((L, DO), jnp.bfloat16),
        ],
        compiler_params=pltpu.CompilerParams(
            vmem_limit_bytes=100 * 1024 * 1024),
    )(pl_mat, px, lx, wpl1, wpl2, b_pl2,
      wlp1, wlp2, b_lp2, g_lp2, be_lp2, g_pl2, be_pl2)

    return (px_p, lx_p)
